# Initial kernel scaffold; baseline (speedup 1.0000x reference)
import functools

import jax
import jax.numpy as jnp
from jax import lax
from jax.experimental import pallas as pl
from jax.experimental.pallas import tpu as pltpu
from jax.experimental.pallas import tpu_sc as plsc

N_REAL = 10000
N_PAD = 10240
N_EDGES = 320000
N_GRAPHS = 512
IN_DIM = 128
H = 64

NC, NS = 2, 16
NW = NC * NS
CHUNK = 128
NCHUNK = 80
EPT = CHUNK * NCHUNK
E_PAD = NW * EPT
ROWS_PER_TILE = N_PAD // NS
PAD_IDX = 10016
DEGW = 16

_MESH = plsc.VectorSubcoreMesh(
    core_axis_name="c", subcore_axis_name="s", num_cores=NC, num_subcores=NS
)


@functools.partial(
    pl.kernel,
    out_type=jax.ShapeDtypeStruct((NC, N_PAD, DEGW), jnp.float32),
    mesh=_MESH,
    scratch_types=[
        pltpu.VMEM((NCHUNK, CHUNK), jnp.int32),
        pltpu.VMEM((CHUNK, DEGW), jnp.float32),
        pltpu.VMEM((ROWS_PER_TILE, DEGW), jnp.float32),
        pltpu.VMEM_SHARED((N_PAD, DEGW), jnp.float32),
    ],
)
def _sc_deg(edst_hbm, out_hbm, didx, buf, stage, degsp):
    c = lax.axis_index("c")
    s = lax.axis_index("s")
    wid = c * NS + s
    pltpu.sync_copy(edst_hbm.at[wid], didx)

    def _zero(i, carry):
        buf[i, :] = jnp.zeros((DEGW,), jnp.float32)
        return carry

    lax.fori_loop(0, CHUNK, _zero, 0)
    r0 = s * ROWS_PER_TILE
    for j in range(ROWS_PER_TILE // CHUNK):
        pltpu.sync_copy(buf, degsp.at[pl.ds(r0 + j * CHUNK, CHUNK)])
    plsc.subcore_barrier()

    def _ones(i, carry):
        buf[i, :] = jnp.ones((DEGW,), jnp.float32)
        return carry

    lax.fori_loop(0, CHUNK, _ones, 0)

    def _scatter(k, carry):
        pltpu.sync_copy(buf, degsp.at[didx.at[k]], add=True)
        return carry

    lax.fori_loop(0, NCHUNK, _scatter, 0)
    plsc.subcore_barrier()
    pltpu.sync_copy(degsp.at[pl.ds(r0, ROWS_PER_TILE)], stage)
    pltpu.sync_copy(stage, out_hbm.at[c, pl.ds(r0, ROWS_PER_TILE)])


@functools.partial(
    pl.kernel,
    out_type=jax.ShapeDtypeStruct((NC, N_PAD, H), jnp.float32),
    mesh=_MESH,
    scratch_types=[
        pltpu.VMEM((NCHUNK, CHUNK), jnp.int32),
        pltpu.VMEM((NCHUNK, CHUNK), jnp.int32),
        pltpu.VMEM((CHUNK, H), jnp.float32),
        pltpu.VMEM((ROWS_PER_TILE, H), jnp.float32),
        pltpu.VMEM_SHARED((N_PAD, H), jnp.float32),
    ],
)
def _sc_agg(xs_hbm, esrc_hbm, edst_hbm, out_hbm, sidx, didx, rows, stage, aggsp):
    c = lax.axis_index("c")
    s = lax.axis_index("s")
    wid = c * NS + s
    pltpu.sync_copy(esrc_hbm.at[wid], sidx)
    pltpu.sync_copy(edst_hbm.at[wid], didx)

    def _zero(i, carry):
        for j in range(H // 16):
            rows[i, pl.ds(j * 16, 16)] = jnp.zeros((16,), jnp.float32)
        return carry

    lax.fori_loop(0, CHUNK, _zero, 0)
    r0 = s * ROWS_PER_TILE
    for j in range(ROWS_PER_TILE // CHUNK):
        pltpu.sync_copy(rows, aggsp.at[pl.ds(r0 + j * CHUNK, CHUNK)])
    plsc.subcore_barrier()

    def _edge(k, carry):
        pltpu.sync_copy(xs_hbm.at[sidx.at[k]], rows)
        pltpu.sync_copy(rows, aggsp.at[didx.at[k]], add=True)
        return carry

    lax.fori_loop(0, NCHUNK, _edge, 0)
    plsc.subcore_barrier()
    pltpu.sync_copy(aggsp.at[pl.ds(r0, ROWS_PER_TILE)], stage)
    pltpu.sync_copy(stage, out_hbm.at[c, pl.ds(r0, ROWS_PER_TILE)])


def _dinv(degp):
    d = degp[0, :, 0:1] + degp[1, :, 0:1] + 1.0
    return lax.rsqrt(d)


def _tc_a_body(v_ref, w1_ref, degp_ref, xs_ref):
    dinv = _dinv(degp_ref[...])
    xw = jnp.dot(v_ref[...], w1_ref[...], preferred_element_type=jnp.float32)
    xs_ref[...] = xw * dinv


def _tc_a(vp, w1, degp):
    return pl.pallas_call(
        _tc_a_body,
        out_shape=jax.ShapeDtypeStruct((N_PAD, H), jnp.float32),
    )(vp, w1, degp)


def _tc_b_body(aggp_ref, xs_ref, degp_ref, b_ref, w_ref, out_ref):
    dinv = _dinv(degp_ref[...])
    agg = aggp_ref[0] + aggp_ref[1]
    h = jnp.maximum(dinv * (agg + xs_ref[...]) + b_ref[...], 0.0)
    out_ref[...] = jnp.dot(h, w_ref[...], preferred_element_type=jnp.float32) * dinv


def _tc_b(aggp, xs, degp, b_row, w):
    return pl.pallas_call(
        _tc_b_body,
        out_shape=jax.ShapeDtypeStruct((N_PAD, H), jnp.float32),
    )(aggp, xs, degp, b_row, w)


BLK = 1024
NB = N_PAD // BLK


def _tc_c_body(aggp_ref, xs_ref, degp_ref, b_ref, bids_ref, wfct_ref, bfc_ref,
               out_ref, sums, cnt):
    i = pl.program_id(0)

    @pl.when(i == 0)
    def _():
        sums[...] = jnp.zeros_like(sums)
        cnt[...] = jnp.zeros_like(cnt)

    dinv = _dinv(degp_ref[...])
    agg = aggp_ref[0] + aggp_ref[1]
    h = jnp.maximum(dinv * (agg + xs_ref[...]) + b_ref[...], 0.0)
    ids = bids_ref[...]
    gi = lax.broadcasted_iota(jnp.int32, (N_GRAPHS, BLK), 0)
    oh = (ids == gi).astype(jnp.float32)
    sums[...] += jnp.dot(oh, h, preferred_element_type=jnp.float32)
    cnt[...] += jnp.sum(oh, axis=1, keepdims=True)

    @pl.when(i == NB - 1)
    def _():
        g = sums[...] / jnp.maximum(cnt[...], 1.0)
        out_ref[...] = (
            jnp.dot(g, wfct_ref[...], preferred_element_type=jnp.float32)
            + bfc_ref[...]
        )


def _tc_c(aggp, xs, degp, b_row, bids, wfct, bfc_row):
    return pl.pallas_call(
        _tc_c_body,
        grid=(NB,),
        in_specs=[
            pl.BlockSpec((NC, BLK, H), lambda i: (0, i, 0)),
            pl.BlockSpec((BLK, H), lambda i: (i, 0)),
            pl.BlockSpec((NC, BLK, DEGW), lambda i: (0, i, 0)),
            pl.BlockSpec((1, H), lambda i: (0, 0)),
            pl.BlockSpec((1, BLK), lambda i: (0, i)),
            pl.BlockSpec((H, H), lambda i: (0, 0)),
            pl.BlockSpec((1, H), lambda i: (0, 0)),
        ],
        out_specs=pl.BlockSpec((N_GRAPHS, H), lambda i: (0, 0)),
        out_shape=jax.ShapeDtypeStruct((N_GRAPHS, H), jnp.float32),
        scratch_shapes=[
            pltpu.VMEM((N_GRAPHS, H), jnp.float32),
            pltpu.VMEM((N_GRAPHS, 1), jnp.float32),
        ],
    )(aggp, xs, degp, b_row, bids, wfct, bfc_row)


def kernel(V, E, batch_ids, W1, b1, W2, b2, Wfc, bfc):
    vp = jnp.zeros((N_PAD, IN_DIM), jnp.float32).at[:N_REAL].set(V)
    e = jnp.full((2, E_PAD), PAD_IDX, jnp.int32).at[:, :N_EDGES].set(E)
    esrc = e[0].reshape(NW, NCHUNK, CHUNK)
    edst = e[1].reshape(NW, NCHUNK, CHUNK)
    bids = jnp.full((1, N_PAD), N_GRAPHS, jnp.int32).at[0, :N_REAL].set(batch_ids)

    degp = _sc_deg(edst)
    xs1 = _tc_a(vp, W1, degp)
    agg1 = _sc_agg(xs1, esrc, edst)
    xs2 = _tc_b(agg1, xs1, degp, b1.reshape(1, H), W2)
    agg2 = _sc_agg(xs2, esrc, edst)
    return _tc_c(agg2, xs2, degp, b2.reshape(1, H), bids, Wfc.T, bfc.reshape(1, H))


# speedup vs baseline: 15.0116x; 15.0116x over previous
import functools

import jax
import jax.numpy as jnp
from jax import lax
from jax.experimental import pallas as pl
from jax.experimental.pallas import tpu as pltpu
from jax.experimental.pallas import tpu_sc as plsc

N_REAL = 10000
N_PAD = 10240
N_EDGES = 320000
N_GRAPHS = 512
IN_DIM = 128
H = 64

NC, NS = 2, 16
NW = NC * NS
CHUNK = 128
NCHUNK = 80
EPT = CHUNK * NCHUNK
E_PAD = NW * EPT
ROWS_PER_TILE = N_PAD // NS
PAD_IDX = 10016
DEGW = 16

_MESH = plsc.VectorSubcoreMesh(
    core_axis_name="c", subcore_axis_name="s", num_cores=NC, num_subcores=NS
)
_SC_PARAMS = pltpu.CompilerParams(use_tc_tiling_on_sc=False)


@functools.partial(
    pl.kernel,
    out_type=jax.ShapeDtypeStruct((NC, N_PAD, DEGW), jnp.float32),
    mesh=_MESH,
    scratch_types=[
        pltpu.VMEM((NCHUNK, CHUNK), jnp.int32),
        pltpu.VMEM((CHUNK, DEGW), jnp.float32),
        pltpu.VMEM((ROWS_PER_TILE, DEGW), jnp.float32),
        pltpu.VMEM_SHARED((N_PAD, DEGW), jnp.float32),
    ],
    compiler_params=_SC_PARAMS,
)
def _sc_deg(edst_hbm, out_hbm, didx, buf, stage, degsp):
    c = lax.axis_index("c")
    s = lax.axis_index("s")
    wid = c * NS + s
    pltpu.sync_copy(edst_hbm.at[wid], didx)

    def _zero(i, carry):
        buf[i, :] = jnp.zeros((DEGW,), jnp.float32)
        return carry

    lax.fori_loop(0, CHUNK, _zero, 0)
    r0 = s * ROWS_PER_TILE
    for j in range(ROWS_PER_TILE // CHUNK):
        pltpu.sync_copy(buf, degsp.at[pl.ds(r0 + j * CHUNK, CHUNK)])
    plsc.subcore_barrier()

    def _ones(i, carry):
        buf[i, :] = jnp.ones((DEGW,), jnp.float32)
        return carry

    lax.fori_loop(0, CHUNK, _ones, 0)

    def _scatter(k, carry):
        pltpu.sync_copy(buf, degsp.at[didx.at[k]], add=True)
        return carry

    lax.fori_loop(0, NCHUNK, _scatter, 0)
    plsc.subcore_barrier()
    pltpu.sync_copy(degsp.at[pl.ds(r0, ROWS_PER_TILE)], stage)
    pltpu.sync_copy(stage, out_hbm.at[c, pl.ds(r0, ROWS_PER_TILE)])


@functools.partial(
    pl.kernel,
    out_type=jax.ShapeDtypeStruct((NC, N_PAD, H), jnp.float32),
    mesh=_MESH,
    scratch_types=[
        pltpu.VMEM((NCHUNK, CHUNK), jnp.int32),
        pltpu.VMEM((NCHUNK, CHUNK), jnp.int32),
        pltpu.VMEM((CHUNK, H), jnp.float32),
        pltpu.VMEM((ROWS_PER_TILE, H), jnp.float32),
        pltpu.VMEM_SHARED((N_PAD, H), jnp.float32),
    ],
    compiler_params=_SC_PARAMS,
)
def _sc_agg(xs_hbm, esrc_hbm, edst_hbm, out_hbm, sidx, didx, rows, stage, aggsp):
    c = lax.axis_index("c")
    s = lax.axis_index("s")
    wid = c * NS + s
    pltpu.sync_copy(esrc_hbm.at[wid], sidx)
    pltpu.sync_copy(edst_hbm.at[wid], didx)

    def _zero(i, carry):
        for j in range(H // 16):
            rows[i, pl.ds(j * 16, 16)] = jnp.zeros((16,), jnp.float32)
        return carry

    lax.fori_loop(0, CHUNK, _zero, 0)
    r0 = s * ROWS_PER_TILE
    for j in range(ROWS_PER_TILE // CHUNK):
        pltpu.sync_copy(rows, aggsp.at[pl.ds(r0 + j * CHUNK, CHUNK)])
    plsc.subcore_barrier()

    def _edge(k, carry):
        pltpu.sync_copy(xs_hbm.at[sidx.at[k]], rows)
        pltpu.sync_copy(rows, aggsp.at[didx.at[k]], add=True)
        return carry

    lax.fori_loop(0, NCHUNK, _edge, 0)
    plsc.subcore_barrier()
    pltpu.sync_copy(aggsp.at[pl.ds(r0, ROWS_PER_TILE)], stage)
    pltpu.sync_copy(stage, out_hbm.at[c, pl.ds(r0, ROWS_PER_TILE)])


def _dinv(degp):
    d = degp[0, :, 0:1] + degp[1, :, 0:1] + 1.0
    return lax.rsqrt(d)


def _tc_a_body(v_ref, w1_ref, degp_ref, xs_ref):
    dinv = _dinv(degp_ref[...])
    xw = jnp.dot(v_ref[...], w1_ref[...], preferred_element_type=jnp.float32)
    xs_ref[...] = xw * dinv


def _tc_a(vp, w1, degp):
    return pl.pallas_call(
        _tc_a_body,
        out_shape=jax.ShapeDtypeStruct((N_PAD, H), jnp.float32),
    )(vp, w1, degp)


def _tc_b_body(aggp_ref, xs_ref, degp_ref, b_ref, w_ref, out_ref):
    dinv = _dinv(degp_ref[...])
    agg = aggp_ref[0] + aggp_ref[1]
    h = jnp.maximum(dinv * (agg + xs_ref[...]) + b_ref[...], 0.0)
    out_ref[...] = jnp.dot(h, w_ref[...], preferred_element_type=jnp.float32) * dinv


def _tc_b(aggp, xs, degp, b_row, w):
    return pl.pallas_call(
        _tc_b_body,
        out_shape=jax.ShapeDtypeStruct((N_PAD, H), jnp.float32),
    )(aggp, xs, degp, b_row, w)


BLK = 1024
NB = N_PAD // BLK


def _tc_c_body(aggp_ref, xs_ref, degp_ref, b_ref, bids_ref, wfct_ref, bfc_ref,
               out_ref, sums, cnt):
    i = pl.program_id(0)

    @pl.when(i == 0)
    def _():
        sums[...] = jnp.zeros_like(sums)
        cnt[...] = jnp.zeros_like(cnt)

    dinv = _dinv(degp_ref[...])
    agg = aggp_ref[0] + aggp_ref[1]
    h = jnp.maximum(dinv * (agg + xs_ref[...]) + b_ref[...], 0.0)
    ids = bids_ref[...]
    gi = lax.broadcasted_iota(jnp.int32, (N_GRAPHS, BLK), 0)
    oh = (ids == gi).astype(jnp.float32)
    sums[...] += jnp.dot(oh, h, preferred_element_type=jnp.float32)
    cnt[...] += jnp.sum(oh, axis=1, keepdims=True)

    @pl.when(i == NB - 1)
    def _():
        g = sums[...] / jnp.maximum(cnt[...], 1.0)
        out_ref[...] = (
            jnp.dot(g, wfct_ref[...], preferred_element_type=jnp.float32)
            + bfc_ref[...]
        )


def _tc_c(aggp, xs, degp, b_row, bids, wfct, bfc_row):
    return pl.pallas_call(
        _tc_c_body,
        grid=(NB,),
        in_specs=[
            pl.BlockSpec((NC, BLK, H), lambda i: (0, i, 0)),
            pl.BlockSpec((BLK, H), lambda i: (i, 0)),
            pl.BlockSpec((NC, BLK, DEGW), lambda i: (0, i, 0)),
            pl.BlockSpec((1, H), lambda i: (0, 0)),
            pl.BlockSpec((1, BLK), lambda i: (0, i)),
            pl.BlockSpec((H, H), lambda i: (0, 0)),
            pl.BlockSpec((1, H), lambda i: (0, 0)),
        ],
        out_specs=pl.BlockSpec((N_GRAPHS, H), lambda i: (0, 0)),
        out_shape=jax.ShapeDtypeStruct((N_GRAPHS, H), jnp.float32),
        scratch_shapes=[
            pltpu.VMEM((N_GRAPHS, H), jnp.float32),
            pltpu.VMEM((N_GRAPHS, 1), jnp.float32),
        ],
    )(aggp, xs, degp, b_row, bids, wfct, bfc_row)


def kernel(V, E, batch_ids, W1, b1, W2, b2, Wfc, bfc):
    vp = jnp.zeros((N_PAD, IN_DIM), jnp.float32).at[:N_REAL].set(V)
    e = jnp.full((2, E_PAD), PAD_IDX, jnp.int32).at[:, :N_EDGES].set(E)
    esrc = e[0].reshape(NW, NCHUNK, CHUNK)
    edst = e[1].reshape(NW, NCHUNK, CHUNK)
    bids = jnp.full((1, N_PAD), N_GRAPHS, jnp.int32).at[0, :N_REAL].set(batch_ids)

    degp = _sc_deg(edst)
    xs1 = _tc_a(vp, W1, degp)
    agg1 = _sc_agg(xs1, esrc, edst)
    xs2 = _tc_b(agg1, xs1, degp, b1.reshape(1, H), W2)
    agg2 = _sc_agg(xs2, esrc, edst)
    return _tc_c(agg2, xs2, degp, b2.reshape(1, H), bids, Wfc.T, bfc.reshape(1, H))


# speedup vs baseline: 17.5904x; 1.1718x over previous
import functools

import jax
import jax.numpy as jnp
from jax import lax
from jax.experimental import pallas as pl
from jax.experimental.pallas import tpu as pltpu
from jax.experimental.pallas import tpu_sc as plsc

N_REAL = 10000
N_PAD = 10240
N_EDGES = 320000
N_GRAPHS = 512
IN_DIM = 128
H = 64

NC, NS = 2, 16
NW = NC * NS
CHUNK = 128
NCHUNK = 80
EPT = CHUNK * NCHUNK
E_PAD = NW * EPT
ROWS_PER_TILE = N_PAD // NS
PAD_IDX = 10016
DEGW = 16

_MESH = plsc.VectorSubcoreMesh(
    core_axis_name="c", subcore_axis_name="s", num_cores=NC, num_subcores=NS
)
_SC_PARAMS = pltpu.CompilerParams(use_tc_tiling_on_sc=False)


@functools.partial(
    pl.kernel,
    out_type=jax.ShapeDtypeStruct((NC, N_PAD, DEGW), jnp.float32),
    mesh=_MESH,
    scratch_types=[
        pltpu.VMEM((NCHUNK, CHUNK), jnp.int32),
        pltpu.VMEM((CHUNK, DEGW), jnp.float32),
        pltpu.VMEM((ROWS_PER_TILE, DEGW), jnp.float32),
        pltpu.VMEM_SHARED((N_PAD, DEGW), jnp.float32),
    ],
    compiler_params=_SC_PARAMS,
)
def _sc_deg(edst_hbm, out_hbm, didx, buf, stage, degsp):
    c = lax.axis_index("c")
    s = lax.axis_index("s")
    wid = c * NS + s
    pltpu.sync_copy(edst_hbm.at[wid], didx)

    def _zero(i, carry):
        buf[i, :] = jnp.zeros((DEGW,), jnp.float32)
        return carry

    lax.fori_loop(0, CHUNK, _zero, 0)
    r0 = s * ROWS_PER_TILE
    for j in range(ROWS_PER_TILE // CHUNK):
        pltpu.sync_copy(buf, degsp.at[pl.ds(r0 + j * CHUNK, CHUNK)])
    plsc.subcore_barrier()

    def _ones(i, carry):
        buf[i, :] = jnp.ones((DEGW,), jnp.float32)
        return carry

    lax.fori_loop(0, CHUNK, _ones, 0)

    def _scatter(k, carry):
        pltpu.sync_copy(buf, degsp.at[didx.at[k]], add=True)
        return carry

    lax.fori_loop(0, NCHUNK, _scatter, 0)
    plsc.subcore_barrier()
    pltpu.sync_copy(degsp.at[pl.ds(r0, ROWS_PER_TILE)], stage)
    pltpu.sync_copy(stage, out_hbm.at[c, pl.ds(r0, ROWS_PER_TILE)])


@functools.partial(
    pl.kernel,
    out_type=jax.ShapeDtypeStruct((NC, N_PAD, H), jnp.float32),
    mesh=_MESH,
    scratch_types=[
        pltpu.VMEM((NCHUNK, CHUNK), jnp.int32),
        pltpu.VMEM((NCHUNK, CHUNK), jnp.int32),
        pltpu.VMEM((CHUNK, H), jnp.float32),
        pltpu.VMEM((CHUNK, H), jnp.float32),
        pltpu.VMEM((ROWS_PER_TILE, H), jnp.float32),
        pltpu.VMEM_SHARED((N_PAD, H), jnp.float32),
        pltpu.SemaphoreType.DMA,
        pltpu.SemaphoreType.DMA,
    ],
    compiler_params=_SC_PARAMS,
)
def _sc_agg(xs_hbm, esrc_hbm, edst_hbm, out_hbm, sidx, didx,
            r0buf, r1buf, stage, aggsp, g0, g1):
    c = lax.axis_index("c")
    s = lax.axis_index("s")
    wid = c * NS + s
    pltpu.sync_copy(esrc_hbm.at[wid], sidx)
    pltpu.sync_copy(edst_hbm.at[wid], didx)
    rows = [r0buf, r1buf]
    gsem = [g0, g1]
    nbuf = 2

    def _zero(i, carry):
        for j in range(H // 16):
            r0buf[i, pl.ds(j * 16, 16)] = jnp.zeros((16,), jnp.float32)
        return carry

    lax.fori_loop(0, CHUNK, _zero, 0)
    r0 = s * ROWS_PER_TILE
    for j in range(ROWS_PER_TILE // CHUNK):
        pltpu.sync_copy(r0buf, aggsp.at[pl.ds(r0 + j * CHUNK, CHUNK)])
    plsc.subcore_barrier()

    for b in range(nbuf):
        pltpu.async_copy(xs_hbm.at[sidx.at[b]], rows[b], gsem[b])

    def _group(k4, carry):
        k = k4 * nbuf
        for b in range(nbuf):
            pltpu.make_async_copy(xs_hbm.at[sidx.at[k + b]], rows[b], gsem[b]).wait()
            pltpu.sync_copy(rows[b], aggsp.at[didx.at[k + b]], add=True)
            pltpu.async_copy(xs_hbm.at[sidx.at[k + b + nbuf]], rows[b], gsem[b])
        return carry

    lax.fori_loop(0, NCHUNK // nbuf - 1, _group, 0)
    ktail = NCHUNK - nbuf
    for b in range(nbuf):
        pltpu.make_async_copy(
            xs_hbm.at[sidx.at[ktail + b]], rows[b], gsem[b]
        ).wait()
        pltpu.sync_copy(rows[b], aggsp.at[didx.at[ktail + b]], add=True)

    plsc.subcore_barrier()
    pltpu.sync_copy(aggsp.at[pl.ds(r0, ROWS_PER_TILE)], stage)
    pltpu.sync_copy(stage, out_hbm.at[c, pl.ds(r0, ROWS_PER_TILE)])


def _dinv(degp):
    d = degp[0, :, 0:1] + degp[1, :, 0:1] + 1.0
    return lax.rsqrt(d)


def _tc_a_body(v_ref, w1_ref, degp_ref, xs_ref):
    dinv = _dinv(degp_ref[...])
    xw = jnp.dot(v_ref[...], w1_ref[...], preferred_element_type=jnp.float32)
    xs_ref[...] = xw * dinv


def _tc_a(vp, w1, degp):
    return pl.pallas_call(
        _tc_a_body,
        out_shape=jax.ShapeDtypeStruct((N_PAD, H), jnp.float32),
    )(vp, w1, degp)


def _tc_b_body(aggp_ref, xs_ref, degp_ref, b_ref, w_ref, out_ref):
    dinv = _dinv(degp_ref[...])
    agg = aggp_ref[0] + aggp_ref[1]
    h = jnp.maximum(dinv * (agg + xs_ref[...]) + b_ref[...], 0.0)
    out_ref[...] = jnp.dot(h, w_ref[...], preferred_element_type=jnp.float32) * dinv


def _tc_b(aggp, xs, degp, b_row, w):
    return pl.pallas_call(
        _tc_b_body,
        out_shape=jax.ShapeDtypeStruct((N_PAD, H), jnp.float32),
    )(aggp, xs, degp, b_row, w)


BLK = 1024
NB = N_PAD // BLK


def _tc_c_body(aggp_ref, xs_ref, degp_ref, b_ref, bids_ref, wfct_ref, bfc_ref,
               out_ref, sums, cnt):
    i = pl.program_id(0)

    @pl.when(i == 0)
    def _():
        sums[...] = jnp.zeros_like(sums)
        cnt[...] = jnp.zeros_like(cnt)

    dinv = _dinv(degp_ref[...])
    agg = aggp_ref[0] + aggp_ref[1]
    h = jnp.maximum(dinv * (agg + xs_ref[...]) + b_ref[...], 0.0)
    ids = bids_ref[...]
    gi = lax.broadcasted_iota(jnp.int32, (N_GRAPHS, BLK), 0)
    oh = (ids == gi).astype(jnp.float32)
    sums[...] += jnp.dot(oh, h, preferred_element_type=jnp.float32)
    cnt[...] += jnp.sum(oh, axis=1, keepdims=True)

    @pl.when(i == NB - 1)
    def _():
        g = sums[...] / jnp.maximum(cnt[...], 1.0)
        out_ref[...] = (
            jnp.dot(g, wfct_ref[...], preferred_element_type=jnp.float32)
            + bfc_ref[...]
        )


def _tc_c(aggp, xs, degp, b_row, bids, wfct, bfc_row):
    return pl.pallas_call(
        _tc_c_body,
        grid=(NB,),
        in_specs=[
            pl.BlockSpec((NC, BLK, H), lambda i: (0, i, 0)),
            pl.BlockSpec((BLK, H), lambda i: (i, 0)),
            pl.BlockSpec((NC, BLK, DEGW), lambda i: (0, i, 0)),
            pl.BlockSpec((1, H), lambda i: (0, 0)),
            pl.BlockSpec((1, BLK), lambda i: (0, i)),
            pl.BlockSpec((H, H), lambda i: (0, 0)),
            pl.BlockSpec((1, H), lambda i: (0, 0)),
        ],
        out_specs=pl.BlockSpec((N_GRAPHS, H), lambda i: (0, 0)),
        out_shape=jax.ShapeDtypeStruct((N_GRAPHS, H), jnp.float32),
        scratch_shapes=[
            pltpu.VMEM((N_GRAPHS, H), jnp.float32),
            pltpu.VMEM((N_GRAPHS, 1), jnp.float32),
        ],
    )(aggp, xs, degp, b_row, bids, wfct, bfc_row)


def kernel(V, E, batch_ids, W1, b1, W2, b2, Wfc, bfc):
    vp = jnp.zeros((N_PAD, IN_DIM), jnp.float32).at[:N_REAL].set(V)
    e = jnp.full((2, E_PAD), PAD_IDX, jnp.int32).at[:, :N_EDGES].set(E)
    esrc = e[0].reshape(NW, NCHUNK, CHUNK)
    edst = e[1].reshape(NW, NCHUNK, CHUNK)
    bids = jnp.full((1, N_PAD), N_GRAPHS, jnp.int32).at[0, :N_REAL].set(batch_ids)

    degp = _sc_deg(edst)
    xs1 = _tc_a(vp, W1, degp)
    agg1 = _sc_agg(xs1, esrc, edst)
    xs2 = _tc_b(agg1, xs1, degp, b1.reshape(1, H), W2)
    agg2 = _sc_agg(xs2, esrc, edst)
    return _tc_c(agg2, xs2, degp, b2.reshape(1, H), bids, Wfc.T, bfc.reshape(1, H))


# speedup vs baseline: 20.0927x; 1.1422x over previous
import functools

import jax
import jax.numpy as jnp
from jax import lax
from jax.experimental import pallas as pl
from jax.experimental.pallas import tpu as pltpu
from jax.experimental.pallas import tpu_sc as plsc

N_REAL = 10000
N_PAD = 10240
N_EDGES = 320000
N_GRAPHS = 512
IN_DIM = 128
H = 64

NC, NS = 2, 16
NW = NC * NS
CHUNK = 128
NCHUNK = 80
NCH_TOT = 2560
SPLIT0 = 120
SPLIT1 = 40
GUARD = SPLIT0 - SPLIT1
E_PAD = (NCH_TOT + GUARD) * CHUNK
ROWS_PER_TILE = N_PAD // NS
PAD_IDX = 10016
DEGW = 16

_MESH = plsc.VectorSubcoreMesh(
    core_axis_name="c", subcore_axis_name="s", num_cores=NC, num_subcores=NS
)
_SC_PARAMS = pltpu.CompilerParams(use_tc_tiling_on_sc=False)


@functools.partial(
    pl.kernel,
    out_type=jax.ShapeDtypeStruct((NC, N_PAD, DEGW), jnp.float32),
    mesh=_MESH,
    scratch_types=[
        pltpu.VMEM((NCHUNK, CHUNK), jnp.int32),
        pltpu.VMEM((CHUNK, DEGW), jnp.float32),
        pltpu.VMEM((ROWS_PER_TILE, DEGW), jnp.float32),
        pltpu.VMEM_SHARED((N_PAD, DEGW), jnp.float32),
    ],
    compiler_params=_SC_PARAMS,
)
def _sc_deg(edst_hbm, out_hbm, didx, buf, stage, degsp):
    c = lax.axis_index("c")
    s = lax.axis_index("s")
    wid = c * NS + s
    pltpu.sync_copy(edst_hbm.at[pl.ds(wid * NCHUNK, NCHUNK)], didx)

    def _zero(i, carry):
        buf[i, :] = jnp.zeros((DEGW,), jnp.float32)
        return carry

    lax.fori_loop(0, CHUNK, _zero, 0)
    r0 = s * ROWS_PER_TILE
    for j in range(ROWS_PER_TILE // CHUNK):
        pltpu.sync_copy(buf, degsp.at[pl.ds(r0 + j * CHUNK, CHUNK)])
    plsc.subcore_barrier()

    def _ones(i, carry):
        buf[i, :] = jnp.ones((DEGW,), jnp.float32)
        return carry

    lax.fori_loop(0, CHUNK, _ones, 0)

    def _scatter(k, carry):
        pltpu.sync_copy(buf, degsp.at[didx.at[k]], add=True)
        return carry

    lax.fori_loop(0, NCHUNK, _scatter, 0)
    plsc.subcore_barrier()
    pltpu.sync_copy(degsp.at[pl.ds(r0, ROWS_PER_TILE)], stage)
    pltpu.sync_copy(stage, out_hbm.at[c, pl.ds(r0, ROWS_PER_TILE)])


@functools.partial(
    pl.kernel,
    out_type=jax.ShapeDtypeStruct((NC, N_PAD, H), jnp.float32),
    mesh=_MESH,
    scratch_types=[
        pltpu.VMEM((SPLIT0, CHUNK), jnp.int32),
        pltpu.VMEM((SPLIT0, CHUNK), jnp.int32),
        pltpu.VMEM((CHUNK, H), jnp.float32),
        pltpu.VMEM((CHUNK, H), jnp.float32),
        pltpu.VMEM((ROWS_PER_TILE, H), jnp.float32),
        pltpu.VMEM_SHARED((N_PAD, H), jnp.float32),
        pltpu.SemaphoreType.DMA,
        pltpu.SemaphoreType.DMA,
    ],
    compiler_params=_SC_PARAMS,
)
def _sc_agg(xs_hbm, esrc_hbm, edst_hbm, out_hbm, sidx, didx,
            r0buf, r1buf, stage, aggsp, g0, g1):
    c = lax.axis_index("c")
    s = lax.axis_index("s")
    start = s * (SPLIT0 + SPLIT1) + jnp.where(c == 0, 0, SPLIT0)
    n_my = jnp.where(c == 0, SPLIT0, SPLIT1)
    pltpu.sync_copy(esrc_hbm.at[pl.ds(start, SPLIT0)], sidx)
    pltpu.sync_copy(edst_hbm.at[pl.ds(start, SPLIT0)], didx)
    rows = [r0buf, r1buf]
    gsem = [g0, g1]
    nbuf = 2

    def _zero(i, carry):
        for j in range(H // 16):
            r0buf[i, pl.ds(j * 16, 16)] = jnp.zeros((16,), jnp.float32)
        return carry

    lax.fori_loop(0, CHUNK, _zero, 0)
    r0 = s * ROWS_PER_TILE
    for j in range(ROWS_PER_TILE // CHUNK):
        pltpu.sync_copy(r0buf, aggsp.at[pl.ds(r0 + j * CHUNK, CHUNK)])
    plsc.subcore_barrier()

    for b in range(nbuf):
        pltpu.async_copy(xs_hbm.at[sidx.at[b]], rows[b], gsem[b])

    def _group(k4, carry):
        k = k4 * nbuf
        for b in range(nbuf):
            pltpu.make_async_copy(xs_hbm.at[sidx.at[k + b]], rows[b], gsem[b]).wait()
            pltpu.sync_copy(rows[b], aggsp.at[didx.at[k + b]], add=True)
            pltpu.async_copy(xs_hbm.at[sidx.at[k + b + nbuf]], rows[b], gsem[b])
        return carry

    lax.fori_loop(0, n_my // nbuf - 1, _group, 0)
    ktail = n_my - nbuf
    for b in range(nbuf):
        pltpu.make_async_copy(
            xs_hbm.at[sidx.at[ktail + b]], rows[b], gsem[b]
        ).wait()
        pltpu.sync_copy(rows[b], aggsp.at[didx.at[ktail + b]], add=True)

    plsc.subcore_barrier()
    pltpu.sync_copy(aggsp.at[pl.ds(r0, ROWS_PER_TILE)], stage)
    pltpu.sync_copy(stage, out_hbm.at[c, pl.ds(r0, ROWS_PER_TILE)])


def _dinv(degp):
    d = degp[0, :, 0:1] + degp[1, :, 0:1] + 1.0
    return lax.rsqrt(d)


def _tc_a_body(v_ref, w1_ref, degp_ref, xs_ref):
    dinv = _dinv(degp_ref[...])
    xw = jnp.dot(v_ref[...], w1_ref[...], preferred_element_type=jnp.float32)
    xs_ref[...] = xw * dinv


def _tc_a(vp, w1, degp):
    return pl.pallas_call(
        _tc_a_body,
        out_shape=jax.ShapeDtypeStruct((N_PAD, H), jnp.float32),
    )(vp, w1, degp)


def _tc_b_body(aggp_ref, xs_ref, degp_ref, b_ref, w_ref, out_ref):
    dinv = _dinv(degp_ref[...])
    agg = aggp_ref[0] + aggp_ref[1]
    h = jnp.maximum(dinv * (agg + xs_ref[...]) + b_ref[...], 0.0)
    out_ref[...] = jnp.dot(h, w_ref[...], preferred_element_type=jnp.float32) * dinv


def _tc_b(aggp, xs, degp, b_row, w):
    return pl.pallas_call(
        _tc_b_body,
        out_shape=jax.ShapeDtypeStruct((N_PAD, H), jnp.float32),
    )(aggp, xs, degp, b_row, w)


BLK = 1024
NB = N_PAD // BLK


def _tc_c_body(aggp_ref, xs_ref, degp_ref, b_ref, bids_ref, wfct_ref, bfc_ref,
               out_ref, sums, cnt):
    i = pl.program_id(0)

    @pl.when(i == 0)
    def _():
        sums[...] = jnp.zeros_like(sums)
        cnt[...] = jnp.zeros_like(cnt)

    dinv = _dinv(degp_ref[...])
    agg = aggp_ref[0] + aggp_ref[1]
    h = jnp.maximum(dinv * (agg + xs_ref[...]) + b_ref[...], 0.0)
    ids = bids_ref[...]
    gi = lax.broadcasted_iota(jnp.int32, (N_GRAPHS, BLK), 0)
    oh = (ids == gi).astype(jnp.float32)
    sums[...] += jnp.dot(oh, h, preferred_element_type=jnp.float32)
    cnt[...] += jnp.sum(oh, axis=1, keepdims=True)

    @pl.when(i == NB - 1)
    def _():
        g = sums[...] / jnp.maximum(cnt[...], 1.0)
        out_ref[...] = (
            jnp.dot(g, wfct_ref[...], preferred_element_type=jnp.float32)
            + bfc_ref[...]
        )


def _tc_c(aggp, xs, degp, b_row, bids, wfct, bfc_row):
    return pl.pallas_call(
        _tc_c_body,
        grid=(NB,),
        in_specs=[
            pl.BlockSpec((NC, BLK, H), lambda i: (0, i, 0)),
            pl.BlockSpec((BLK, H), lambda i: (i, 0)),
            pl.BlockSpec((NC, BLK, DEGW), lambda i: (0, i, 0)),
            pl.BlockSpec((1, H), lambda i: (0, 0)),
            pl.BlockSpec((1, BLK), lambda i: (0, i)),
            pl.BlockSpec((H, H), lambda i: (0, 0)),
            pl.BlockSpec((1, H), lambda i: (0, 0)),
        ],
        out_specs=pl.BlockSpec((N_GRAPHS, H), lambda i: (0, 0)),
        out_shape=jax.ShapeDtypeStruct((N_GRAPHS, H), jnp.float32),
        scratch_shapes=[
            pltpu.VMEM((N_GRAPHS, H), jnp.float32),
            pltpu.VMEM((N_GRAPHS, 1), jnp.float32),
        ],
    )(aggp, xs, degp, b_row, bids, wfct, bfc_row)


def kernel(V, E, batch_ids, W1, b1, W2, b2, Wfc, bfc):
    vp = jnp.zeros((N_PAD, IN_DIM), jnp.float32).at[:N_REAL].set(V)
    e = jnp.full((2, E_PAD), PAD_IDX, jnp.int32).at[:, :N_EDGES].set(E)
    esrc = e[0].reshape(NCH_TOT + GUARD, CHUNK)
    edst = e[1].reshape(NCH_TOT + GUARD, CHUNK)
    bids = jnp.full((1, N_PAD), N_GRAPHS, jnp.int32).at[0, :N_REAL].set(batch_ids)

    degp = _sc_deg(edst)
    xs1 = _tc_a(vp, W1, degp)
    agg1 = _sc_agg(xs1, esrc, edst)
    xs2 = _tc_b(agg1, xs1, degp, b1.reshape(1, H), W2)
    agg2 = _sc_agg(xs2, esrc, edst)
    return _tc_c(agg2, xs2, degp, b2.reshape(1, H), bids, Wfc.T, bfc.reshape(1, H))
